# baseline (speedup 1.0000x reference)
import jax
import jax.numpy as jnp
from jax.experimental import pallas as pl

_NODES = 82
_HID = 30
_G_BLOCK = 95


def _fused_kernel(x_ref, wz_ref, wh_ref, bz_ref, bh_ref, w2_ref, b2_ref, o_ref):
    xb = x_ref[0]
    a = jnp.dot(xb, wz_ref[...], preferred_element_type=jnp.float32) + bz_ref[...]
    c = jnp.dot(xb, wh_ref[...], preferred_element_type=jnp.float32) + bh_ref[...]
    h = jnp.maximum((1.0 - jax.nn.sigmoid(a)) * jnp.tanh(c), 0.0)
    hp = jnp.pad(h, ((0, 0), (0, 128 - _HID)))
    hf = hp.reshape(_G_BLOCK, _NODES * 128)
    o_ref[...] = (
        jnp.dot(hf, w2_ref[...], preferred_element_type=jnp.float32) + b2_ref[...]
    )[None]


@jax.jit
def kernel(x, edge_index, Wxz, bxz, Whz, bhz, Wxr, bxr, Whr, bhr, Wxh, bxh, Whh, bhh, W_lin, b_lin):
    n, d = x.shape
    hid = Wxz.shape[1]
    g = n // _NODES
    feat = hid * _NODES
    out_dim = W_lin.shape[0]
    bz = (bxz + bhz).reshape(1, hid)
    bh = (bxh + bhh).reshape(1, hid)
    w2 = jnp.pad(
        W_lin.T.reshape(_NODES, hid, out_dim), ((0, 0), (0, 128 - hid), (0, 0))
    ).reshape(_NODES * 128, out_dim)

    rows = _G_BLOCK * _NODES
    n_blocks = g // _G_BLOCK
    x3 = x.reshape(n_blocks, rows, d)
    out = pl.pallas_call(
        _fused_kernel,
        grid=(n_blocks,),
        in_specs=[
            pl.BlockSpec((1, rows, d), lambda i: (i, 0, 0)),
            pl.BlockSpec((d, hid), lambda i: (0, 0)),
            pl.BlockSpec((d, hid), lambda i: (0, 0)),
            pl.BlockSpec((1, hid), lambda i: (0, 0)),
            pl.BlockSpec((1, hid), lambda i: (0, 0)),
            pl.BlockSpec((_NODES * 128, out_dim), lambda i: (0, 0)),
            pl.BlockSpec((1, out_dim), lambda i: (0, 0)),
        ],
        out_specs=pl.BlockSpec((1, _G_BLOCK, out_dim), lambda i: (i, 0, 0)),
        out_shape=jax.ShapeDtypeStruct((n_blocks, _G_BLOCK, out_dim), jnp.float32),
    )(x3, Wxz, Wxh, bz, bh, w2, b_lin.reshape(1, out_dim))
    return out.reshape(g, out_dim)


# speedup vs baseline: 1.0149x; 1.0149x over previous
import jax
import jax.numpy as jnp
from jax.experimental import pallas as pl

_NUM_NODES_PER_GRAPH = 82
_ROW_BLOCK = 1984


def _gate_kernel(x_ref, wz_ref, wh_ref, bz_ref, bh_ref, o_ref):
    xb = x_ref[...]
    a = jnp.dot(xb, wz_ref[...], preferred_element_type=jnp.float32) + bz_ref[...]
    c = jnp.dot(xb, wh_ref[...], preferred_element_type=jnp.float32) + bh_ref[...]
    h = (1.0 - jax.nn.sigmoid(a)) * jnp.tanh(c)
    o_ref[...] = jnp.maximum(h, 0.0)


def _head_kernel(h_ref, w_ref, b_ref, o_ref):
    o_ref[...] = (
        jnp.dot(h_ref[...], w_ref[...], preferred_element_type=jnp.float32)
        + b_ref[...]
    )


@jax.jit
def kernel(x, edge_index, Wxz, bxz, Whz, bhz, Wxr, bxr, Whr, bhr, Wxh, bxh, Whh, bhh, W_lin, b_lin):
    n, d = x.shape
    hid = Wxz.shape[1]
    bz = (bxz + bhz).reshape(1, hid)
    bh = (bxh + bhh).reshape(1, hid)

    grid = pl.cdiv(n, _ROW_BLOCK)
    h = pl.pallas_call(
        _gate_kernel,
        grid=(grid,),
        in_specs=[
            pl.BlockSpec((_ROW_BLOCK, d), lambda i: (i, 0)),
            pl.BlockSpec((d, hid), lambda i: (0, 0)),
            pl.BlockSpec((d, hid), lambda i: (0, 0)),
            pl.BlockSpec((1, hid), lambda i: (0, 0)),
            pl.BlockSpec((1, hid), lambda i: (0, 0)),
        ],
        out_specs=pl.BlockSpec((_ROW_BLOCK, hid), lambda i: (i, 0)),
        out_shape=jax.ShapeDtypeStruct((n, hid), jnp.float32),
    )(x, Wxz, Wxh, bz, bh)

    feat = hid * _NUM_NODES_PER_GRAPH
    g = n // _NUM_NODES_PER_GRAPH
    hf = h.reshape(g, feat)
    w2 = W_lin.T
    out_dim = w2.shape[1]
    out = pl.pallas_call(
        _head_kernel,
        grid=(1,),
        in_specs=[
            pl.BlockSpec((g, feat), lambda i: (0, 0)),
            pl.BlockSpec((feat, out_dim), lambda i: (0, 0)),
            pl.BlockSpec((1, out_dim), lambda i: (0, 0)),
        ],
        out_specs=pl.BlockSpec((g, out_dim), lambda i: (0, 0)),
        out_shape=jax.ShapeDtypeStruct((g, out_dim), jnp.float32),
    )(hf, w2, b_lin.reshape(1, out_dim))
    return out


# speedup vs baseline: 1.1950x; 1.1774x over previous
import jax
import jax.numpy as jnp
from jax.experimental import pallas as pl

_NUM_NODES_PER_GRAPH = 82
_ROW_BLOCK = 7936


def _gate_kernel(x_ref, wz_ref, wh_ref, bz_ref, bh_ref, o_ref):
    xb = x_ref[...]
    a = jnp.dot(xb, wz_ref[...], preferred_element_type=jnp.float32) + bz_ref[...]
    c = jnp.dot(xb, wh_ref[...], preferred_element_type=jnp.float32) + bh_ref[...]
    h = (1.0 - jax.nn.sigmoid(a)) * jnp.tanh(c)
    o_ref[...] = jnp.maximum(h, 0.0)


def _head_kernel(h_ref, w_ref, b_ref, o_ref):
    o_ref[...] = (
        jnp.dot(h_ref[...], w_ref[...], preferred_element_type=jnp.float32)
        + b_ref[...]
    )


@jax.jit
def kernel(x, edge_index, Wxz, bxz, Whz, bhz, Wxr, bxr, Whr, bhr, Wxh, bxh, Whh, bhh, W_lin, b_lin):
    n, d = x.shape
    hid = Wxz.shape[1]
    bz = (bxz + bhz).reshape(1, hid)
    bh = (bxh + bhh).reshape(1, hid)

    grid = pl.cdiv(n, _ROW_BLOCK)
    h = pl.pallas_call(
        _gate_kernel,
        grid=(grid,),
        in_specs=[
            pl.BlockSpec((_ROW_BLOCK, d), lambda i: (i, 0)),
            pl.BlockSpec((d, hid), lambda i: (0, 0)),
            pl.BlockSpec((d, hid), lambda i: (0, 0)),
            pl.BlockSpec((1, hid), lambda i: (0, 0)),
            pl.BlockSpec((1, hid), lambda i: (0, 0)),
        ],
        out_specs=pl.BlockSpec((_ROW_BLOCK, hid), lambda i: (i, 0)),
        out_shape=jax.ShapeDtypeStruct((n, hid), jnp.float32),
    )(x, Wxz, Wxh, bz, bh)

    feat = hid * _NUM_NODES_PER_GRAPH
    g = n // _NUM_NODES_PER_GRAPH
    hf = h.reshape(g, feat)
    w2 = W_lin.T
    out_dim = w2.shape[1]
    out = pl.pallas_call(
        _head_kernel,
        grid=(1,),
        in_specs=[
            pl.BlockSpec((g, feat), lambda i: (0, 0)),
            pl.BlockSpec((feat, out_dim), lambda i: (0, 0)),
            pl.BlockSpec((1, out_dim), lambda i: (0, 0)),
        ],
        out_specs=pl.BlockSpec((g, out_dim), lambda i: (0, 0)),
        out_shape=jax.ShapeDtypeStruct((g, out_dim), jnp.float32),
    )(hf, w2, b_lin.reshape(1, out_dim))
    return out


# speedup vs baseline: 1.2003x; 1.0045x over previous
import jax
import jax.numpy as jnp
from jax.experimental import pallas as pl

_NUM_NODES_PER_GRAPH = 82
_ROW_BLOCK = 13056


def _gate_kernel(x_ref, wz_ref, wh_ref, bz_ref, bh_ref, o_ref):
    xb = x_ref[...]
    a = jnp.dot(xb, wz_ref[...], preferred_element_type=jnp.float32) + bz_ref[...]
    c = jnp.dot(xb, wh_ref[...], preferred_element_type=jnp.float32) + bh_ref[...]
    h = (1.0 - jax.nn.sigmoid(a)) * jnp.tanh(c)
    o_ref[...] = jnp.maximum(h, 0.0)


def _head_kernel(h_ref, w_ref, b_ref, o_ref):
    o_ref[...] = (
        jnp.dot(h_ref[...], w_ref[...], preferred_element_type=jnp.float32)
        + b_ref[...]
    )


@jax.jit
def kernel(x, edge_index, Wxz, bxz, Whz, bhz, Wxr, bxr, Whr, bhr, Wxh, bxh, Whh, bhh, W_lin, b_lin):
    n, d = x.shape
    hid = Wxz.shape[1]
    bz = (bxz + bhz).reshape(1, hid)
    bh = (bxh + bhh).reshape(1, hid)

    grid = pl.cdiv(n, _ROW_BLOCK)
    h = pl.pallas_call(
        _gate_kernel,
        grid=(grid,),
        in_specs=[
            pl.BlockSpec((_ROW_BLOCK, d), lambda i: (i, 0)),
            pl.BlockSpec((d, hid), lambda i: (0, 0)),
            pl.BlockSpec((d, hid), lambda i: (0, 0)),
            pl.BlockSpec((1, hid), lambda i: (0, 0)),
            pl.BlockSpec((1, hid), lambda i: (0, 0)),
        ],
        out_specs=pl.BlockSpec((_ROW_BLOCK, hid), lambda i: (i, 0)),
        out_shape=jax.ShapeDtypeStruct((n, hid), jnp.float32),
    )(x, Wxz, Wxh, bz, bh)

    feat = hid * _NUM_NODES_PER_GRAPH
    g = n // _NUM_NODES_PER_GRAPH
    hf = h.reshape(g, feat)
    w2 = W_lin.T
    out_dim = w2.shape[1]
    out = pl.pallas_call(
        _head_kernel,
        grid=(1,),
        in_specs=[
            pl.BlockSpec((g, feat), lambda i: (0, 0)),
            pl.BlockSpec((feat, out_dim), lambda i: (0, 0)),
            pl.BlockSpec((1, out_dim), lambda i: (0, 0)),
        ],
        out_specs=pl.BlockSpec((g, out_dim), lambda i: (0, 0)),
        out_shape=jax.ShapeDtypeStruct((g, out_dim), jnp.float32),
    )(hf, w2, b_lin.reshape(1, out_dim))
    return out


# speedup vs baseline: 1.3631x; 1.1356x over previous
import jax
import jax.numpy as jnp
from jax.experimental import pallas as pl

_NUM_NODES_PER_GRAPH = 82
_ROW_BLOCK = 13056


def _gate_kernel(x_ref, wz_ref, wh_ref, bz_ref, bh_ref, o_ref):
    xb = x_ref[...]
    a = jnp.dot(xb, wz_ref[...], preferred_element_type=jnp.float32) + bz_ref[...]
    c = jnp.dot(xb, wh_ref[...], preferred_element_type=jnp.float32) + bh_ref[...]
    h = (1.0 - jax.nn.sigmoid(a)) * jnp.tanh(c)
    o_ref[...] = jnp.maximum(h, 0.0).astype(jnp.bfloat16)


def _head_kernel(h_ref, w_ref, b_ref, o_ref):
    o_ref[...] = (
        jnp.dot(h_ref[...], w_ref[...], preferred_element_type=jnp.float32)
        + b_ref[...]
    )


@jax.jit
def kernel(x, edge_index, Wxz, bxz, Whz, bhz, Wxr, bxr, Whr, bhr, Wxh, bxh, Whh, bhh, W_lin, b_lin):
    n, d = x.shape
    hid = Wxz.shape[1]
    bz = (bxz + bhz).reshape(1, hid)
    bh = (bxh + bhh).reshape(1, hid)

    grid = pl.cdiv(n, _ROW_BLOCK)
    h = pl.pallas_call(
        _gate_kernel,
        grid=(grid,),
        in_specs=[
            pl.BlockSpec((_ROW_BLOCK, d), lambda i: (i, 0)),
            pl.BlockSpec((d, hid), lambda i: (0, 0)),
            pl.BlockSpec((d, hid), lambda i: (0, 0)),
            pl.BlockSpec((1, hid), lambda i: (0, 0)),
            pl.BlockSpec((1, hid), lambda i: (0, 0)),
        ],
        out_specs=pl.BlockSpec((_ROW_BLOCK, hid), lambda i: (i, 0)),
        out_shape=jax.ShapeDtypeStruct((n, hid), jnp.bfloat16),
    )(x, Wxz, Wxh, bz, bh)

    feat = hid * _NUM_NODES_PER_GRAPH
    g = n // _NUM_NODES_PER_GRAPH
    hf = h.reshape(g, feat)
    w2 = W_lin.T.astype(jnp.bfloat16)
    out_dim = w2.shape[1]
    out = pl.pallas_call(
        _head_kernel,
        grid=(1,),
        in_specs=[
            pl.BlockSpec((g, feat), lambda i: (0, 0)),
            pl.BlockSpec((feat, out_dim), lambda i: (0, 0)),
            pl.BlockSpec((1, out_dim), lambda i: (0, 0)),
        ],
        out_specs=pl.BlockSpec((g, out_dim), lambda i: (0, 0)),
        out_shape=jax.ShapeDtypeStruct((g, out_dim), jnp.float32),
    )(hf, w2, b_lin.reshape(1, out_dim))
    return out
